# baseline (speedup 1.0000x reference)
import functools

import jax
import jax.numpy as jnp
from jax import lax
from jax.experimental import pallas as pl
from jax.experimental.pallas import tpu as pltpu
from jax.experimental.pallas import tpu_sc as plsc

_NUM_CORES = 2
_NUM_SUBCORES = 16
_NUM_WORKERS = _NUM_CORES * _NUM_SUBCORES


def _tr_body(x_ref, o_ref):
    o_ref[...] = x_ref[...].T


def _transpose_w(w):
    d, v = w.shape
    blk = 4096
    return pl.pallas_call(
        _tr_body,
        grid=(pl.cdiv(v, blk),),
        in_specs=[pl.BlockSpec((d, blk), lambda i: (0, i))],
        out_specs=pl.BlockSpec((blk, d), lambda i: (i, 0)),
        out_shape=jax.ShapeDtypeStruct((v, d), jnp.float32),
    )(w)


def _transpose_g(g):
    b, d = g.shape
    blk = 4096
    return pl.pallas_call(
        _tr_body,
        grid=(b // blk,),
        in_specs=[pl.BlockSpec((blk, d), lambda i: (i, 0))],
        out_specs=pl.BlockSpec((d, blk), lambda i: (0, i)),
        out_shape=jax.ShapeDtypeStruct((d, b), jnp.float32),
    )(g)


_CHUNK = 128


def _sc_gather(table, idx3):
    v, d = table.shape
    nw, n_chunks, chunk = idx3.shape
    b = nw * n_chunks * chunk
    b_per_w = n_chunks * chunk

    mesh = plsc.VectorSubcoreMesh(
        core_axis_name="c",
        subcore_axis_name="s",
        num_cores=_NUM_CORES,
        num_subcores=_NUM_SUBCORES,
    )

    @functools.partial(
        pl.kernel,
        mesh=mesh,
        compiler_params=pltpu.CompilerParams(use_tc_tiling_on_sc=False),
        out_type=jax.ShapeDtypeStruct((b, d), jnp.float32),
        scratch_types=[
            pltpu.VMEM((n_chunks, chunk), jnp.int32),
            pltpu.VMEM((chunk, d), jnp.float32),
            pltpu.SemaphoreType.DMA,
        ],
    )
    def gather_kernel(table_hbm, idx_hbm, out_hbm, idx_v, rows_v, sem):
        wid = lax.axis_index("s") * _NUM_CORES + lax.axis_index("c")
        base = wid * b_per_w
        pltpu.sync_copy(idx_hbm.at[wid], idx_v)

        def body(g, carry):
            pltpu.async_copy(table_hbm.at[idx_v.at[g]], rows_v, sem).wait()
            pltpu.sync_copy(rows_v, out_hbm.at[pl.ds(base + g * chunk, chunk)])
            return carry

        lax.fori_loop(0, n_chunks, body, 0)

    return gather_kernel(table, idx3)


def kernel(vocab_idx, W):
    d, _ = W.shape
    s0, s1 = vocab_idx.shape
    b = s0 * s1
    idx3 = vocab_idx.reshape(_NUM_WORKERS, b // (_NUM_WORKERS * _CHUNK), _CHUNK)
    idx3 = idx3.astype(jnp.int32)
    t = _transpose_w(W)
    g = _sc_gather(t, idx3)
    out = _transpose_g(g)
    return out.reshape(d, s0, s1)


# speedup vs baseline: 1.1524x; 1.1524x over previous
import functools

import jax
import jax.numpy as jnp
from jax import lax
from jax.experimental import pallas as pl
from jax.experimental.pallas import tpu as pltpu
from jax.experimental.pallas import tpu_sc as plsc

_NUM_CORES = 2
_NUM_SUBCORES = 16
_NUM_WORKERS = _NUM_CORES * _NUM_SUBCORES

_LANE = 16
_NJ = 4


def _tr_body(x_ref, o_ref):
    o_ref[:, 0:64] = x_ref[...].T


def _transpose_w(w):
    d, v = w.shape
    blk = 4096
    return pl.pallas_call(
        _tr_body,
        grid=(pl.cdiv(v, blk),),
        in_specs=[pl.BlockSpec((d, blk), lambda i: (0, i))],
        out_specs=pl.BlockSpec((blk, 2 * d), lambda i: (i, 0)),
        out_shape=jax.ShapeDtypeStruct((v, 2 * d), jnp.float32),
    )(w)


def _sc_gather_t(table, idx_t):
    v, d = table.shape
    nj_total, ni = idx_t.shape
    ni_w = ni // _NUM_WORKERS
    n_groups = nj_total // _NJ

    mesh = plsc.VectorSubcoreMesh(
        core_axis_name="c",
        subcore_axis_name="s",
        num_cores=_NUM_CORES,
        num_subcores=_NUM_SUBCORES,
    )

    @functools.partial(
        pl.kernel,
        mesh=mesh,
        compiler_params=pltpu.CompilerParams(
            use_tc_tiling_on_sc=False, needs_layout_passes=False
        ),
        out_type=jax.ShapeDtypeStruct((d, nj_total, ni), jnp.float32),
        scratch_types=[
            pltpu.VMEM((nj_total, ni_w), jnp.int32),
            pltpu.VMEM((ni_w, d), jnp.float32),
            pltpu.VMEM((ni_w, d), jnp.float32),
            pltpu.VMEM((d, ni_w), jnp.float32),
            pltpu.SemaphoreType.DMA,
        ],
    )
    def gather_kernel(t_hbm, idx_hbm, out_hbm, idx_v, g_v0, g_v1, p_v, sem):
        wid = lax.axis_index("s") * _NUM_CORES + lax.axis_index("c")
        i0 = wid * ni_w
        pltpu.sync_copy(idx_hbm.at[:, pl.ds(i0, ni_w)], idx_v)

        def fire(j, gref):
            pltpu.async_copy(t_hbm.at[idx_v.at[j]], gref, sem)

        def drain(j, gref):
            pltpu.make_async_copy(t_hbm.at[idx_v.at[j]], gref, sem).wait()

        def transpose_store(j, gref):
            for dd in range(d):
                for k in range(ni_w // _LANE):
                    rows = lax.iota(jnp.int32, _LANE) + (k * _LANE)
                    cols = jnp.full((_LANE,), dd, jnp.int32)
                    p_v[dd, pl.ds(k * _LANE, _LANE)] = plsc.load_gather(
                        gref, [rows, cols]
                    )
            pltpu.sync_copy(p_v, out_hbm.at[:, j, pl.ds(i0, ni_w)])

        fire(0, g_v0)
        fire(1, g_v1)

        def body(h, carry):
            j0 = 2 * h
            drain(j0, g_v0)
            transpose_store(j0, g_v0)

            @pl.when(j0 + 2 < nj_total)
            def _():
                fire(j0 + 2, g_v0)

            drain(j0 + 1, g_v1)
            transpose_store(j0 + 1, g_v1)

            @pl.when(j0 + 3 < nj_total)
            def _():
                fire(j0 + 3, g_v1)

            return carry

        lax.fori_loop(0, nj_total // 2, body, 0)

    return gather_kernel(table, idx_t)


def kernel(vocab_idx, W):
    d, v = W.shape
    s0, s1 = vocab_idx.shape
    idx_t = vocab_idx.T.astype(jnp.int32) * 2
    t2 = _transpose_w(W)
    t = t2.reshape(2 * v, d)
    p = _sc_gather_t(t, idx_t)
    return jnp.swapaxes(p, 1, 2)


# speedup vs baseline: 2.6592x; 2.3076x over previous
import functools

import jax
import jax.numpy as jnp
from jax import lax
from jax.experimental import pallas as pl
from jax.experimental.pallas import tpu as pltpu
from jax.experimental.pallas import tpu_sc as plsc

_NUM_CORES = 2
_NUM_SUBCORES = 16
_NUM_WORKERS = _NUM_CORES * _NUM_SUBCORES

_CHUNK = 128
_JP = 100


def _tr_body(x_ref, o_ref):
    o_ref[:, 0:64] = x_ref[...].T


def _transpose_w(w):
    d, v = w.shape
    blk = 4096
    return pl.pallas_call(
        _tr_body,
        grid=(pl.cdiv(v, blk),),
        in_specs=[pl.BlockSpec((d, blk), lambda i: (0, i))],
        out_specs=pl.BlockSpec((blk, 2 * d), lambda i: (i, 0)),
        out_shape=jax.ShapeDtypeStruct((v, 2 * d), jnp.float32),
    )(w)


def _sc_gather(table, idx3):
    v2, d = table.shape
    nw, n_chunks, chunk = idx3.shape
    b = nw * n_chunks * chunk
    b_per_w = n_chunks * chunk

    mesh = plsc.VectorSubcoreMesh(
        core_axis_name="c",
        subcore_axis_name="s",
        num_cores=_NUM_CORES,
        num_subcores=_NUM_SUBCORES,
    )

    @functools.partial(
        pl.kernel,
        mesh=mesh,
        compiler_params=pltpu.CompilerParams(
            use_tc_tiling_on_sc=False, needs_layout_passes=False
        ),
        out_type=jax.ShapeDtypeStruct((b, d), jnp.float32),
        scratch_types=[
            pltpu.VMEM((n_chunks, chunk), jnp.int32),
            pltpu.VMEM((chunk, d), jnp.float32),
            pltpu.VMEM((chunk, d), jnp.float32),
            pltpu.SemaphoreType.DMA,
        ],
    )
    def gather_kernel(table_hbm, idx_hbm, out_hbm, idx_v, g_v0, g_v1, sem):
        wid = lax.axis_index("s") * _NUM_CORES + lax.axis_index("c")
        base = wid * b_per_w
        pltpu.sync_copy(idx_hbm.at[wid], idx_v)

        def fire(g, buf):
            pltpu.async_copy(table_hbm.at[idx_v.at[g]], buf, sem)

        def drain(g, buf):
            pltpu.make_async_copy(table_hbm.at[idx_v.at[g]], buf, sem).wait()

        def store(g, buf):
            pltpu.sync_copy(buf, out_hbm.at[pl.ds(base + g * chunk, chunk)])

        fire(0, g_v0)
        fire(1, g_v1)

        def body(h, carry):
            g0 = 2 * h
            drain(g0, g_v0)

            @pl.when(g0 + 2 < n_chunks)
            def _():
                fire(g0 + 2, g_v0)

            store(g0, g_v0)
            drain(g0 + 1, g_v1)

            @pl.when(g0 + 3 < n_chunks)
            def _():
                fire(g0 + 3, g_v1)

            store(g0 + 1, g_v1)
            return carry

        lax.fori_loop(0, n_chunks // 2, body, 0)

    return gather_kernel(table, idx3)


def _trg_body(x_ref, o_ref):
    for jj in range(8):
        for u in range(16):
            xt = x_ref[jj, pl.ds(128 * u, 128), :].T
            o_ref[:, jj, pl.ds(128 * u, 128)] = xt[0:64, :]
            o_ref[:, jj, pl.ds(2048 + 128 * u, 128)] = xt[64:128, :]


def _transpose_g(g3):
    nj, nm, _ = g3.shape
    ni = 2 * nm
    return pl.pallas_call(
        _trg_body,
        grid=(nj // 8,),
        in_specs=[pl.BlockSpec((8, nm, 128), lambda j: (j, 0, 0))],
        out_specs=pl.BlockSpec((64, 8, ni), lambda j: (0, j, 0)),
        out_shape=jax.ShapeDtypeStruct((64, nj, ni), jnp.float32),
    )(g3)


def kernel(vocab_idx, W):
    d, v = W.shape
    s0, s1 = vocab_idx.shape
    b = s0 * s1
    n_chunks = b // (_NUM_WORKERS * _CHUNK)
    idx_r = (
        (vocab_idx.T.astype(jnp.int32) * 2)
        .reshape(s1, 2, s0 // 2)
        .swapaxes(1, 2)
    )
    idx3 = idx_r.reshape(_NUM_WORKERS, n_chunks, _CHUNK)
    t = _transpose_w(W).reshape(2 * v, d)
    g = _sc_gather(t, idx3)
    g3 = g.reshape(s1, s0 // 2, 2 * d)
    p = _transpose_g(g3)
    return jnp.swapaxes(p, 1, 2)


# speedup vs baseline: 3.5051x; 1.3181x over previous
import functools

import jax
import jax.numpy as jnp
from jax import lax
from jax.experimental import pallas as pl
from jax.experimental.pallas import tpu as pltpu
from jax.experimental.pallas import tpu_sc as plsc

_NUM_CORES = 2
_NUM_SUBCORES = 16
_NUM_WORKERS = _NUM_CORES * _NUM_SUBCORES

_CHUNK = 128
_JP = 100


def _tr_body(x_ref, o_ref):
    o_ref[:, 0:64] = x_ref[...].T


def _transpose_w(w):
    d, v = w.shape
    blk = 4096
    return pl.pallas_call(
        _tr_body,
        grid=(pl.cdiv(v, blk),),
        in_specs=[pl.BlockSpec((d, blk), lambda i: (0, i))],
        out_specs=pl.BlockSpec((blk, 2 * d), lambda i: (i, 0)),
        out_shape=jax.ShapeDtypeStruct((v, 2 * d), jnp.float32),
    )(w)


def _sc_gather(table, idx2d):
    v2, d = table.shape
    ni, nj = idx2d.shape
    ni_w = ni // _NUM_WORKERS

    mesh = plsc.VectorSubcoreMesh(
        core_axis_name="c",
        subcore_axis_name="s",
        num_cores=_NUM_CORES,
        num_subcores=_NUM_SUBCORES,
    )

    @functools.partial(
        pl.kernel,
        mesh=mesh,
        compiler_params=pltpu.CompilerParams(
            use_tc_tiling_on_sc=False, needs_layout_passes=False
        ),
        out_type=jax.ShapeDtypeStruct((nj, ni // 2, 2 * d), jnp.float32),
        scratch_types=[
            pltpu.VMEM((ni_w, nj), jnp.int32),
            pltpu.VMEM((nj, ni_w), jnp.int32),
            pltpu.VMEM((ni_w, d), jnp.float32),
            pltpu.VMEM((ni_w, d), jnp.float32),
            pltpu.SemaphoreType.DMA,
        ],
    )
    def gather_kernel(
        table_hbm, idx_hbm, out_hbm, idx_vt, idx_v, g_v0, g_v1, sem
    ):
        wid = lax.axis_index("s") * _NUM_CORES + lax.axis_index("c")
        i0 = wid * ni_w
        half = ni // 2

        pltpu.sync_copy(idx_hbm.at[pl.ds(i0, ni_w)], idx_vt)
        for j in range(nj):
            cols = jnp.full((16,), j, jnp.int32)
            for k in range(ni_w // 16):
                rows = lax.iota(jnp.int32, 16) + (k * 16)
                idx_v[j, pl.ds(k * 16, 16)] = plsc.load_gather(
                    idx_vt, [rows, cols]
                )

        def fire(j, buf):
            pltpu.async_copy(table_hbm.at[idx_v.at[j]], buf, sem)

        def drain(j, buf):
            pltpu.make_async_copy(table_hbm.at[idx_v.at[j]], buf, sem).wait()

        def store(j, buf):
            pltpu.sync_copy(
                buf,
                out_hbm.at[
                    j,
                    pl.ds(lax.rem(i0, half), ni_w),
                    pl.ds((i0 // half) * d, d),
                ],
            )

        fire(0, g_v0)
        fire(1, g_v1)

        def body(h, carry):
            j0 = 2 * h
            drain(j0, g_v0)

            @pl.when(j0 + 2 < nj)
            def _():
                fire(j0 + 2, g_v0)

            store(j0, g_v0)
            drain(j0 + 1, g_v1)

            @pl.when(j0 + 3 < nj)
            def _():
                fire(j0 + 3, g_v1)

            store(j0 + 1, g_v1)
            return carry

        lax.fori_loop(0, nj // 2, body, 0)

    return gather_kernel(table, idx2d)


def _trg_body(x_ref, o_ref):
    for jj in range(8):
        for u in range(16):
            xt = x_ref[jj, pl.ds(128 * u, 128), :].T
            o_ref[:, jj, pl.ds(128 * u, 128)] = xt[0:64, :]
            o_ref[:, jj, pl.ds(2048 + 128 * u, 128)] = xt[64:128, :]


def _transpose_g(g3):
    nj, nm, _ = g3.shape
    ni = 2 * nm
    return pl.pallas_call(
        _trg_body,
        grid=(nj // 8,),
        in_specs=[pl.BlockSpec((8, nm, 128), lambda j: (j, 0, 0))],
        out_specs=pl.BlockSpec((64, 8, ni), lambda j: (0, j, 0)),
        out_shape=jax.ShapeDtypeStruct((64, nj, ni), jnp.float32),
    )(g3)


def kernel(vocab_idx, W):
    d, v = W.shape
    s0, s1 = vocab_idx.shape
    idx2d = vocab_idx.astype(jnp.int32) * 2
    t = _transpose_w(W).reshape(2 * v, d)
    g3 = _sc_gather(t, idx2d)
    p = _transpose_g(g3)
    return jnp.swapaxes(p, 1, 2)


# speedup vs baseline: 3.9153x; 1.1170x over previous
import functools

import jax
import jax.numpy as jnp
from jax import lax
from jax.experimental import pallas as pl
from jax.experimental.pallas import tpu as pltpu
from jax.experimental.pallas import tpu_sc as plsc

_NUM_CORES = 2
_NUM_SUBCORES = 16
_NUM_WORKERS = _NUM_CORES * _NUM_SUBCORES

_CHUNK = 128
_JP = 100


def _tr_body(x_ref, o_ref):
    o_ref[:, 0:64] = x_ref[...].T


def _transpose_w(w):
    d, v = w.shape
    blk = 8192
    return pl.pallas_call(
        _tr_body,
        grid=(pl.cdiv(v, blk),),
        in_specs=[pl.BlockSpec((d, blk), lambda i: (0, i))],
        out_specs=pl.BlockSpec((blk, 2 * d), lambda i: (i, 0)),
        out_shape=jax.ShapeDtypeStruct((v, 2 * d), jnp.float32),
    )(w)


def _sc_gather(table, idx2d):
    v2, d = table.shape
    ni, nj = idx2d.shape
    ni_w = ni // _NUM_WORKERS

    mesh = plsc.VectorSubcoreMesh(
        core_axis_name="c",
        subcore_axis_name="s",
        num_cores=_NUM_CORES,
        num_subcores=_NUM_SUBCORES,
    )

    @functools.partial(
        pl.kernel,
        mesh=mesh,
        compiler_params=pltpu.CompilerParams(
            use_tc_tiling_on_sc=False, needs_layout_passes=False
        ),
        out_type=jax.ShapeDtypeStruct((nj, ni // 2, 2 * d), jnp.float32),
        scratch_types=[
            pltpu.VMEM((ni_w, nj), jnp.int32),
            pltpu.VMEM((nj, ni_w), jnp.int32),
            pltpu.VMEM((ni_w, d), jnp.float32),
            pltpu.VMEM((ni_w, d), jnp.float32),
            pltpu.SemaphoreType.DMA,
        ],
    )
    def gather_kernel(
        table_hbm, idx_hbm, out_hbm, idx_vt, idx_v, g_v0, g_v1, sem
    ):
        wid = lax.axis_index("s") * _NUM_CORES + lax.axis_index("c")
        i0 = wid * ni_w
        half = ni // 2

        pltpu.sync_copy(idx_hbm.at[pl.ds(i0, ni_w)], idx_vt)
        for j in range(nj):
            cols = jnp.full((16,), j, jnp.int32)
            for k in range(ni_w // 16):
                rows = lax.iota(jnp.int32, 16) + (k * 16)
                idx_v[j, pl.ds(k * 16, 16)] = plsc.load_gather(
                    idx_vt, [rows, cols]
                )

        def fire(j, buf):
            pltpu.async_copy(table_hbm.at[idx_v.at[j]], buf, sem)

        def drain(j, buf):
            pltpu.make_async_copy(table_hbm.at[idx_v.at[j]], buf, sem).wait()

        def store(j, buf):
            pltpu.sync_copy(
                buf,
                out_hbm.at[
                    j,
                    pl.ds(lax.rem(i0, half), ni_w),
                    pl.ds((i0 // half) * d, d),
                ],
            )

        fire(0, g_v0)
        fire(1, g_v1)

        def body(h, carry):
            j0 = 2 * h
            drain(j0, g_v0)

            @pl.when(j0 + 2 < nj)
            def _():
                fire(j0 + 2, g_v0)

            store(j0, g_v0)
            drain(j0 + 1, g_v1)

            @pl.when(j0 + 3 < nj)
            def _():
                fire(j0 + 3, g_v1)

            store(j0 + 1, g_v1)
            return carry

        lax.fori_loop(0, nj // 2, body, 0)

    return gather_kernel(table, idx2d)


def _trg_body(x_ref, o_ref):
    for jj in range(8):
        for u in range(16):
            xt = x_ref[jj, pl.ds(128 * u, 128), :].T
            o_ref[:, jj, pl.ds(128 * u, 128)] = xt[0:64, :]
            o_ref[:, jj, pl.ds(2048 + 128 * u, 128)] = xt[64:128, :]


def _transpose_g(g3):
    nj, nm, _ = g3.shape
    ni = 2 * nm
    return pl.pallas_call(
        _trg_body,
        grid=(nj // 8,),
        in_specs=[pl.BlockSpec((8, nm, 128), lambda j: (j, 0, 0))],
        out_specs=pl.BlockSpec((64, 8, ni), lambda j: (0, j, 0)),
        out_shape=jax.ShapeDtypeStruct((64, nj, ni), jnp.float32),
    )(g3)


def kernel(vocab_idx, W):
    d, v = W.shape
    s0, s1 = vocab_idx.shape
    idx2d = vocab_idx.astype(jnp.int32) * 2
    t = _transpose_w(W).reshape(2 * v, d)
    g3 = _sc_gather(t, idx2d)
    p = _transpose_g(g3)
    return jnp.swapaxes(p, 1, 2)
